# baseline (speedup 1.0000x reference)
import functools

import jax
import jax.numpy as jnp
from jax import lax
from jax.experimental import pallas as pl
from jax.experimental.pallas import tpu as pltpu
from jax.experimental.pallas import tpu_sc as plsc

_T, _D, _E, _K, _DFF, _DSH = 2048, 2048, 64, 2, 512, 512
_BLK = 32
_NB = 192
_SMAX = _NB * _BLK
_TB = 128
_NW = 32


def _router_body(x_ref, gw_ref, inds_ref, sc_ref, rank_ref, cnt_ref, run_ref):
    pid = pl.program_id(0)

    @pl.when(pid == 0)
    def _():
        run_ref[...] = jnp.zeros_like(run_ref)

    xb = x_ref[...]
    logits = lax.dot_general(xb, gw_ref[...], (((1,), (1,)), ((), ())),
                             preferred_element_type=jnp.float32)
    m = jnp.max(logits, axis=1, keepdims=True)
    ex = jnp.exp(logits - m)
    p = ex / jnp.sum(ex, axis=1, keepdims=True)
    idx = lax.broadcasted_iota(jnp.int32, p.shape, 1)
    v1 = jnp.max(p, axis=1, keepdims=True)
    i1 = jnp.min(jnp.where(p == v1, idx, _E), axis=1)
    pm = jnp.where(idx == i1[:, None], -jnp.inf, p)
    v2 = jnp.max(pm, axis=1, keepdims=True)
    i2 = jnp.min(jnp.where(pm == v2, idx, _E), axis=1)
    s = v1[:, 0] + v2[:, 0]
    inds_ref[...] = jnp.stack([i1, i2], axis=0)
    sc_ref[...] = jnp.stack([v1[:, 0] / s, v2[:, 0] / s], axis=0)

    eq1 = (idx == i1[:, None]).astype(jnp.float32)
    eq2 = (idx == i2[:, None]).astype(jnp.float32)
    both = eq1 + eq2
    r = lax.broadcasted_iota(jnp.int32, (_TB, _TB), 0)
    c = lax.broadcasted_iota(jnp.int32, (_TB, _TB), 1)
    tri = jnp.where(c < r, 1.0, 0.0).astype(jnp.float32)
    pre = lax.dot_general(tri, both, (((1,), (0,)), ((), ())),
                          preferred_element_type=jnp.float32)
    tot = pre + run_ref[...]
    rank1 = jnp.sum(eq1 * tot, axis=1)
    rank2 = jnp.sum(eq2 * tot, axis=1)
    rank_ref[...] = jnp.stack([rank1, rank2], axis=0).astype(jnp.int32)
    run_new = run_ref[...] + jnp.sum(both, axis=0, keepdims=True)
    run_ref[...] = run_new
    cnt_ref[...] = run_new.astype(jnp.int32)


def _router(x, gate_w):
    return pl.pallas_call(
        _router_body,
        grid=(_T // _TB,),
        in_specs=[
            pl.BlockSpec((_TB, _D), lambda i: (i, 0)),
            pl.BlockSpec((_E, _D), lambda i: (0, 0)),
        ],
        out_specs=[
            pl.BlockSpec((_K, _TB), lambda i: (0, i)),
            pl.BlockSpec((_K, _TB), lambda i: (0, i)),
            pl.BlockSpec((_K, _TB), lambda i: (0, i)),
            pl.BlockSpec((1, _E), lambda i: (0, 0)),
        ],
        out_shape=[
            jax.ShapeDtypeStruct((_K, _T), jnp.int32),
            jax.ShapeDtypeStruct((_K, _T), jnp.float32),
            jax.ShapeDtypeStruct((_K, _T), jnp.int32),
            jax.ShapeDtypeStruct((1, _E), jnp.int32),
        ],
        scratch_shapes=[pltpu.VMEM((1, _E), jnp.float32)],
    )(x, gate_w)


@functools.lru_cache(maxsize=None)
def _make_sc_scatter(chunk):
    per_w = _T // _NW
    n_steps = per_w // chunk
    assert n_steps % 2 == 0
    mesh = plsc.VectorSubcoreMesh(core_axis_name="c", subcore_axis_name="s")

    def body(x_ref, i0_ref, i1_ref, out_ref, idx_v, rows_v,
             sem_l0, sem_l1, sem_s0, sem_s1):
        wid = lax.axis_index("s") * 2 + lax.axis_index("c")
        base = wid * per_w
        sem_l = (sem_l0, sem_l1)
        sem_s = (sem_s0, sem_s1)

        def start_load(i, b):
            off = base + i * chunk
            pltpu.sync_copy(i0_ref.at[pl.ds(off, chunk)], idx_v.at[2 * b])
            pltpu.sync_copy(i1_ref.at[pl.ds(off, chunk)], idx_v.at[2 * b + 1])
            pltpu.async_copy(x_ref.at[pl.ds(off, chunk)], rows_v.at[b],
                             sem_l[b])

        def wait_load(b):
            pltpu.make_async_copy(x_ref.at[pl.ds(base, chunk)], rows_v.at[b],
                                  sem_l[b]).wait()

        def start_scat(b):
            pltpu.async_copy(rows_v.at[b], out_ref.at[idx_v.at[2 * b]],
                             sem_s[b])
            pltpu.async_copy(rows_v.at[b], out_ref.at[idx_v.at[2 * b + 1]],
                             sem_s[b])

        def wait_scat(b):
            pltpu.make_async_copy(rows_v.at[b], out_ref.at[idx_v.at[2 * b]],
                                  sem_s[b]).wait()
            pltpu.make_async_copy(rows_v.at[b], out_ref.at[idx_v.at[2 * b + 1]],
                                  sem_s[b]).wait()

        start_load(0, 0)

        def outer(o, carry):
            for b in (0, 1):
                i = o * 2 + b
                wait_load(b)
                start_scat(b)
                nb = 1 - b

                @pl.when(i + 1 < n_steps)
                def _():
                    @pl.when(i >= 1)
                    def _():
                        wait_scat(nb)
                    start_load(i + 1, nb)
            return carry

        lax.fori_loop(0, n_steps // 2, outer, 0)
        wait_scat(0)
        wait_scat(1)

    return pl.kernel(
        body,
        out_type=jax.ShapeDtypeStruct((_SMAX, _D), jnp.float32),
        mesh=mesh,
        scratch_types=[
            pltpu.VMEM((4, chunk), jnp.int32),
            pltpu.VMEM((2, chunk, _D), jnp.float32),
            pltpu.SemaphoreType.DMA,
            pltpu.SemaphoreType.DMA,
            pltpu.SemaphoreType.DMA,
            pltpu.SemaphoreType.DMA,
        ],
    )


def _scatter_x(x, idx0, idx1):
    return _make_sc_scatter(16)(x, idx0, idx1)


@functools.lru_cache(maxsize=None)
def _make_sc_gather(n_out, width, chunk, dtype):
    per_w = n_out // _NW
    n_steps = per_w // chunk
    assert n_steps % 2 == 0
    mesh = plsc.VectorSubcoreMesh(core_axis_name="c", subcore_axis_name="s")

    def body(table_ref, idx_ref, out_ref, idx_v, rows_v,
             sem_g0, sem_g1, sem_o0, sem_o1):
        wid = lax.axis_index("s") * 2 + lax.axis_index("c")
        base = wid * per_w
        sem_g = (sem_g0, sem_g1)
        sem_o = (sem_o0, sem_o1)

        def start_gather(i, b):
            pltpu.sync_copy(idx_ref.at[pl.ds(base + i * chunk, chunk)],
                            idx_v.at[b])
            pltpu.async_copy(table_ref.at[idx_v.at[b]], rows_v.at[b],
                             sem_g[b])

        def wait_gather(b):
            pltpu.make_async_copy(table_ref.at[idx_v.at[b]], rows_v.at[b],
                                  sem_g[b]).wait()

        def start_out(i, b):
            pltpu.async_copy(rows_v.at[b],
                             out_ref.at[pl.ds(base + i * chunk, chunk)],
                             sem_o[b])

        def wait_out(b):
            pltpu.make_async_copy(rows_v.at[b],
                                  out_ref.at[pl.ds(base, chunk)],
                                  sem_o[b]).wait()

        start_gather(0, 0)

        def outer(o, carry):
            for b in (0, 1):
                i = o * 2 + b
                wait_gather(b)
                start_out(i, b)
                nb = 1 - b

                @pl.when(i + 1 < n_steps)
                def _():
                    @pl.when(i >= 1)
                    def _():
                        wait_out(nb)
                    start_gather(i + 1, nb)
            return carry

        lax.fori_loop(0, n_steps // 2, outer, 0)
        wait_out(0)
        wait_out(1)

    return pl.kernel(
        body,
        out_type=jax.ShapeDtypeStruct((n_out, width), dtype),
        mesh=mesh,
        scratch_types=[
            pltpu.VMEM((2, chunk), jnp.int32),
            pltpu.VMEM((2, chunk, width), dtype),
            pltpu.SemaphoreType.DMA,
            pltpu.SemaphoreType.DMA,
            pltpu.SemaphoreType.DMA,
            pltpu.SemaphoreType.DMA,
        ],
    )


def _gather_y(table, idx):
    return _make_sc_gather(_T * _K, _D, 16, jnp.float32)(table, idx)


def _gmm_body(be_ref, nv_ref, x_ref, wg_ref, wu_ref, wd_ref, ws_ref, out_ref):
    @pl.when(pl.program_id(0) < nv_ref[0])
    def _():
        xb = x_ref[...]
        g = lax.dot_general(xb, wg_ref[0], (((1,), (1,)), ((), ())),
                            preferred_element_type=jnp.float32)
        u = lax.dot_general(xb, wu_ref[0], (((1,), (1,)), ((), ())),
                            preferred_element_type=jnp.float32)
        h = g * jax.nn.sigmoid(g) * u
        o = lax.dot_general(h, wd_ref[0], (((1,), (1,)), ((), ())),
                            preferred_element_type=jnp.float32)
        out_ref[...] = o * ws_ref[0]


def _gmm(blk_expert, nv, x_sorted, Wg, Wu, Wd, ws3):
    def _c(i, nv_ref):
        return jnp.minimum(i, nv_ref[0] - 1)

    grid_spec = pltpu.PrefetchScalarGridSpec(
        num_scalar_prefetch=2,
        grid=(_NB,),
        in_specs=[
            pl.BlockSpec((_BLK, _D), lambda i, be, nv: (_c(i, nv), 0)),
            pl.BlockSpec((1, _DFF, _D),
                         lambda i, be, nv: (be[_c(i, nv)], 0, 0)),
            pl.BlockSpec((1, _DFF, _D),
                         lambda i, be, nv: (be[_c(i, nv)], 0, 0)),
            pl.BlockSpec((1, _D, _DFF),
                         lambda i, be, nv: (be[_c(i, nv)], 0, 0)),
            pl.BlockSpec((1, _BLK, 1), lambda i, be, nv: (_c(i, nv), 0, 0)),
        ],
        out_specs=pl.BlockSpec((_BLK, _D), lambda i, be, nv: (_c(i, nv), 0)),
    )
    return pl.pallas_call(
        _gmm_body,
        grid_spec=grid_spec,
        out_shape=jax.ShapeDtypeStruct((_SMAX, _D), jnp.float32),
    )(blk_expert, nv, x_sorted, Wg, Wu, Wd, ws3)


def _combine_body(x_ref, yg_ref, sg_ref, su_ref, sd_ref, se_ref, out_ref):
    xb = x_ref[...]
    g = lax.dot_general(xb, sg_ref[...], (((1,), (1,)), ((), ())),
                        preferred_element_type=jnp.float32)
    u = lax.dot_general(xb, su_ref[...], (((1,), (1,)), ((), ())),
                        preferred_element_type=jnp.float32)
    sh = g * jax.nn.sigmoid(g) * u
    sy = lax.dot_general(sh, sd_ref[...], (((1,), (1,)), ((), ())),
                         preferred_element_type=jnp.float32)
    gate = jax.nn.sigmoid(
        lax.dot_general(xb, se_ref[...], (((1,), (1,)), ((), ())),
                        preferred_element_type=jnp.float32))
    yg = yg_ref[...]
    out_ref[...] = yg[:, 0, :] + yg[:, 1, :] + gate * sy


def _combine(x, yg3, sg_w, su_w, sd_w, seg_w):
    return pl.pallas_call(
        _combine_body,
        grid=(_T // _TB,),
        in_specs=[
            pl.BlockSpec((_TB, _D), lambda i: (i, 0)),
            pl.BlockSpec((_TB, _K, _D), lambda i: (i, 0, 0)),
            pl.BlockSpec((_DSH, _D), lambda i: (0, 0)),
            pl.BlockSpec((_DSH, _D), lambda i: (0, 0)),
            pl.BlockSpec((_D, _DSH), lambda i: (0, 0)),
            pl.BlockSpec((1, _D), lambda i: (0, 0)),
        ],
        out_specs=pl.BlockSpec((_TB, _D), lambda i: (i, 0)),
        out_shape=jax.ShapeDtypeStruct((_T, _D), jnp.float32),
    )(x, yg3, sg_w, su_w, sd_w, seg_w)


def _dispatch(inds2, sc2, rank2, cnt):
    flat_e = inds2.T.reshape(-1)
    flat_s = sc2.T.reshape(-1)
    rank_flat = rank2.T.reshape(-1)
    counts = cnt[0]
    pcnt = ((counts + _BLK - 1) // _BLK) * _BLK
    pend = jnp.cumsum(pcnt)
    poff = pend - pcnt
    slot = poff[flat_e] + rank_flat
    ws_flat = jnp.zeros((_SMAX,), jnp.float32).at[slot].set(flat_s)
    starts = jnp.arange(_NB, dtype=jnp.int32) * _BLK
    blk_expert = jnp.clip(
        jnp.searchsorted(pend, starts, side="right"), 0, _E - 1
    ).astype(jnp.int32)
    slot2 = slot.reshape(_T, _K)
    nv = (pend[_E - 1] // _BLK).astype(jnp.int32).reshape(1)
    return slot, slot2[:, 0], slot2[:, 1], ws_flat, blk_expert, nv


def kernel(x, gate_w, Wg, Wu, Wd, sg_w, su_w, sd_w, seg_w):
    inds2, sc2, rank2, cnt = _router(x, gate_w)
    slot, idx0, idx1, ws_flat, blk_expert, nv = _dispatch(inds2, sc2, rank2,
                                                          cnt)
    x_sorted = _scatter_x(x, idx0, idx1)
    y_sorted = _gmm(blk_expert, nv, x_sorted, Wg, Wu, Wd,
                    ws_flat.reshape(_NB, _BLK, 1))
    yg = _gather_y(y_sorted, slot)
    return _combine(x, yg.reshape(_T, _K, _D), sg_w, su_w, sd_w, seg_w)


# speedup vs baseline: 1.3293x; 1.3293x over previous
import functools

import jax
import jax.numpy as jnp
from jax import lax
from jax.experimental import pallas as pl
from jax.experimental.pallas import tpu as pltpu
from jax.experimental.pallas import tpu_sc as plsc

_T, _D, _E, _K, _DFF, _DSH = 2048, 2048, 64, 2, 512, 512
_BLK = 64
_NB = 128
_SMAX = _NB * _BLK
_TB = 256
_NW = 32


def _router_body(x_ref, gw_ref, inds_ref, sc_ref, rank_ref, cnt_ref, run_ref):
    pid = pl.program_id(0)

    @pl.when(pid == 0)
    def _():
        run_ref[...] = jnp.zeros_like(run_ref)

    xb = x_ref[...]
    logits = lax.dot_general(xb, gw_ref[...], (((1,), (1,)), ((), ())),
                             preferred_element_type=jnp.float32)
    m = jnp.max(logits, axis=1, keepdims=True)
    ex = jnp.exp(logits - m)
    p = ex / jnp.sum(ex, axis=1, keepdims=True)
    idx = lax.broadcasted_iota(jnp.int32, p.shape, 1)
    v1 = jnp.max(p, axis=1, keepdims=True)
    i1 = jnp.min(jnp.where(p == v1, idx, _E), axis=1)
    pm = jnp.where(idx == i1[:, None], -jnp.inf, p)
    v2 = jnp.max(pm, axis=1, keepdims=True)
    i2 = jnp.min(jnp.where(pm == v2, idx, _E), axis=1)
    s = v1[:, 0] + v2[:, 0]
    inds_ref[...] = jnp.stack([i1, i2], axis=0)
    sc_ref[...] = jnp.stack([v1[:, 0] / s, v2[:, 0] / s], axis=0)

    eq1 = (idx == i1[:, None]).astype(jnp.float32)
    eq2 = (idx == i2[:, None]).astype(jnp.float32)
    both = eq1 + eq2
    r = lax.broadcasted_iota(jnp.int32, (_TB, _TB), 0)
    c = lax.broadcasted_iota(jnp.int32, (_TB, _TB), 1)
    tri = jnp.where(c < r, 1.0, 0.0).astype(jnp.float32)
    pre = lax.dot_general(tri, both, (((1,), (0,)), ((), ())),
                          preferred_element_type=jnp.float32)
    tot = pre + run_ref[...]
    rank1 = jnp.sum(eq1 * tot, axis=1)
    rank2 = jnp.sum(eq2 * tot, axis=1)
    rank_ref[...] = jnp.stack([rank1, rank2], axis=0).astype(jnp.int32)
    run_new = run_ref[...] + jnp.sum(both, axis=0, keepdims=True)
    run_ref[...] = run_new
    cnt_ref[...] = run_new.astype(jnp.int32)


def _router(x, gate_w):
    return pl.pallas_call(
        _router_body,
        grid=(_T // _TB,),
        in_specs=[
            pl.BlockSpec((_TB, _D), lambda i: (i, 0)),
            pl.BlockSpec((_E, _D), lambda i: (0, 0)),
        ],
        out_specs=[
            pl.BlockSpec((_K, _TB), lambda i: (0, i)),
            pl.BlockSpec((_K, _TB), lambda i: (0, i)),
            pl.BlockSpec((_K, _TB), lambda i: (0, i)),
            pl.BlockSpec((1, _E), lambda i: (0, 0)),
        ],
        out_shape=[
            jax.ShapeDtypeStruct((_K, _T), jnp.int32),
            jax.ShapeDtypeStruct((_K, _T), jnp.float32),
            jax.ShapeDtypeStruct((_K, _T), jnp.int32),
            jax.ShapeDtypeStruct((1, _E), jnp.int32),
        ],
        scratch_shapes=[pltpu.VMEM((1, _E), jnp.float32)],
    )(x, gate_w)


@functools.lru_cache(maxsize=None)
def _make_sc_scatter(chunk):
    per_w = _T // _NW
    n_steps = per_w // chunk
    assert n_steps % 2 == 0
    mesh = plsc.VectorSubcoreMesh(core_axis_name="c", subcore_axis_name="s")

    def body(x_ref, i0_ref, i1_ref, out_ref, idx_v, rows_v,
             sem_l0, sem_l1, sem_s0, sem_s1):
        wid = lax.axis_index("s") * 2 + lax.axis_index("c")
        base = wid * per_w
        sem_l = (sem_l0, sem_l1)
        sem_s = (sem_s0, sem_s1)

        def start_load(i, b):
            off = base + i * chunk
            pltpu.sync_copy(i0_ref.at[pl.ds(off, chunk)], idx_v.at[2 * b])
            pltpu.sync_copy(i1_ref.at[pl.ds(off, chunk)], idx_v.at[2 * b + 1])
            pltpu.async_copy(x_ref.at[pl.ds(off, chunk)], rows_v.at[b],
                             sem_l[b])

        def wait_load(b):
            pltpu.make_async_copy(x_ref.at[pl.ds(base, chunk)], rows_v.at[b],
                                  sem_l[b]).wait()

        def start_scat(b):
            pltpu.async_copy(rows_v.at[b], out_ref.at[idx_v.at[2 * b]],
                             sem_s[b])
            pltpu.async_copy(rows_v.at[b], out_ref.at[idx_v.at[2 * b + 1]],
                             sem_s[b])

        def wait_scat(b):
            pltpu.make_async_copy(rows_v.at[b], out_ref.at[idx_v.at[2 * b]],
                                  sem_s[b]).wait()
            pltpu.make_async_copy(rows_v.at[b], out_ref.at[idx_v.at[2 * b + 1]],
                                  sem_s[b]).wait()

        start_load(0, 0)

        def outer(o, carry):
            for b in (0, 1):
                i = o * 2 + b
                wait_load(b)
                start_scat(b)
                nb = 1 - b

                @pl.when(i + 1 < n_steps)
                def _():
                    @pl.when(i >= 1)
                    def _():
                        wait_scat(nb)
                    start_load(i + 1, nb)
            return carry

        lax.fori_loop(0, n_steps // 2, outer, 0)
        wait_scat(0)
        wait_scat(1)

    return pl.kernel(
        body,
        out_type=jax.ShapeDtypeStruct((_SMAX, _D), jnp.float32),
        mesh=mesh,
        scratch_types=[
            pltpu.VMEM((4, chunk), jnp.int32),
            pltpu.VMEM((2, chunk, _D), jnp.float32),
            pltpu.SemaphoreType.DMA,
            pltpu.SemaphoreType.DMA,
            pltpu.SemaphoreType.DMA,
            pltpu.SemaphoreType.DMA,
        ],
    )


def _scatter_x(x, idx0, idx1):
    return _make_sc_scatter(16)(x, idx0, idx1)


@functools.lru_cache(maxsize=None)
def _make_sc_gather(n_out, width, chunk, dtype):
    per_w = n_out // _NW
    n_steps = per_w // chunk
    assert n_steps % 2 == 0
    mesh = plsc.VectorSubcoreMesh(core_axis_name="c", subcore_axis_name="s")

    def body(table_ref, idx_ref, out_ref, idx_v, rows_v,
             sem_g0, sem_g1, sem_o0, sem_o1):
        wid = lax.axis_index("s") * 2 + lax.axis_index("c")
        base = wid * per_w
        sem_g = (sem_g0, sem_g1)
        sem_o = (sem_o0, sem_o1)

        def start_gather(i, b):
            pltpu.sync_copy(idx_ref.at[pl.ds(base + i * chunk, chunk)],
                            idx_v.at[b])
            pltpu.async_copy(table_ref.at[idx_v.at[b]], rows_v.at[b],
                             sem_g[b])

        def wait_gather(b):
            pltpu.make_async_copy(table_ref.at[idx_v.at[b]], rows_v.at[b],
                                  sem_g[b]).wait()

        def start_out(i, b):
            pltpu.async_copy(rows_v.at[b],
                             out_ref.at[pl.ds(base + i * chunk, chunk)],
                             sem_o[b])

        def wait_out(b):
            pltpu.make_async_copy(rows_v.at[b],
                                  out_ref.at[pl.ds(base, chunk)],
                                  sem_o[b]).wait()

        start_gather(0, 0)

        def outer(o, carry):
            for b in (0, 1):
                i = o * 2 + b
                wait_gather(b)
                start_out(i, b)
                nb = 1 - b

                @pl.when(i + 1 < n_steps)
                def _():
                    @pl.when(i >= 1)
                    def _():
                        wait_out(nb)
                    start_gather(i + 1, nb)
            return carry

        lax.fori_loop(0, n_steps // 2, outer, 0)
        wait_out(0)
        wait_out(1)

    return pl.kernel(
        body,
        out_type=jax.ShapeDtypeStruct((n_out, width), dtype),
        mesh=mesh,
        scratch_types=[
            pltpu.VMEM((2, chunk), jnp.int32),
            pltpu.VMEM((2, chunk, width), dtype),
            pltpu.SemaphoreType.DMA,
            pltpu.SemaphoreType.DMA,
            pltpu.SemaphoreType.DMA,
            pltpu.SemaphoreType.DMA,
        ],
    )


def _gather_y(table, idx):
    return _make_sc_gather(_T * _K, _D, 16, jnp.float32)(table, idx)


def _gmm_body(be_ref, nv_ref, x_ref, wg_ref, wu_ref, wd_ref, ws_ref, out_ref):
    @pl.when(pl.program_id(0) < nv_ref[0])
    def _():
        xb = x_ref[...]
        g = lax.dot_general(xb, wg_ref[0], (((1,), (1,)), ((), ())),
                            preferred_element_type=jnp.float32)
        u = lax.dot_general(xb, wu_ref[0], (((1,), (1,)), ((), ())),
                            preferred_element_type=jnp.float32)
        h = g * jax.nn.sigmoid(g) * u
        o = lax.dot_general(h, wd_ref[0], (((1,), (1,)), ((), ())),
                            preferred_element_type=jnp.float32)
        out_ref[...] = o * ws_ref[0]


def _gmm(blk_expert, nv, x_sorted, Wg, Wu, Wd, ws3):
    def _c(i, nv_ref):
        return jnp.minimum(i, nv_ref[0] - 1)

    grid_spec = pltpu.PrefetchScalarGridSpec(
        num_scalar_prefetch=2,
        grid=(_NB,),
        in_specs=[
            pl.BlockSpec((_BLK, _D), lambda i, be, nv: (_c(i, nv), 0)),
            pl.BlockSpec((1, _DFF, _D),
                         lambda i, be, nv: (be[_c(i, nv)], 0, 0)),
            pl.BlockSpec((1, _DFF, _D),
                         lambda i, be, nv: (be[_c(i, nv)], 0, 0)),
            pl.BlockSpec((1, _D, _DFF),
                         lambda i, be, nv: (be[_c(i, nv)], 0, 0)),
            pl.BlockSpec((1, _BLK, 1), lambda i, be, nv: (_c(i, nv), 0, 0)),
        ],
        out_specs=pl.BlockSpec((_BLK, _D), lambda i, be, nv: (_c(i, nv), 0)),
    )
    return pl.pallas_call(
        _gmm_body,
        grid_spec=grid_spec,
        out_shape=jax.ShapeDtypeStruct((_SMAX, _D), jnp.float32),
    )(blk_expert, nv, x_sorted, Wg, Wu, Wd, ws3)


def _combine_body(x_ref, yg_ref, sg_ref, su_ref, sd_ref, se_ref, out_ref):
    xb = x_ref[...]
    g = lax.dot_general(xb, sg_ref[...], (((1,), (1,)), ((), ())),
                        preferred_element_type=jnp.float32)
    u = lax.dot_general(xb, su_ref[...], (((1,), (1,)), ((), ())),
                        preferred_element_type=jnp.float32)
    sh = g * jax.nn.sigmoid(g) * u
    sy = lax.dot_general(sh, sd_ref[...], (((1,), (1,)), ((), ())),
                         preferred_element_type=jnp.float32)
    gate = jax.nn.sigmoid(
        lax.dot_general(xb, se_ref[...], (((1,), (1,)), ((), ())),
                        preferred_element_type=jnp.float32))
    yg = yg_ref[...]
    out_ref[...] = yg[:, 0, :] + yg[:, 1, :] + gate * sy


def _combine(x, yg3, sg_w, su_w, sd_w, seg_w):
    return pl.pallas_call(
        _combine_body,
        grid=(_T // _TB,),
        in_specs=[
            pl.BlockSpec((_TB, _D), lambda i: (i, 0)),
            pl.BlockSpec((_TB, _K, _D), lambda i: (i, 0, 0)),
            pl.BlockSpec((_DSH, _D), lambda i: (0, 0)),
            pl.BlockSpec((_DSH, _D), lambda i: (0, 0)),
            pl.BlockSpec((_D, _DSH), lambda i: (0, 0)),
            pl.BlockSpec((1, _D), lambda i: (0, 0)),
        ],
        out_specs=pl.BlockSpec((_TB, _D), lambda i: (i, 0)),
        out_shape=jax.ShapeDtypeStruct((_T, _D), jnp.float32),
    )(x, yg3, sg_w, su_w, sd_w, seg_w)


def _dispatch(inds2, sc2, rank2, cnt):
    flat_e = inds2.T.reshape(-1)
    flat_s = sc2.T.reshape(-1)
    rank_flat = rank2.T.reshape(-1)
    counts = cnt[0]
    pcnt = ((counts + _BLK - 1) // _BLK) * _BLK
    pend = jnp.cumsum(pcnt)
    poff = pend - pcnt
    slot = poff[flat_e] + rank_flat
    ws_flat = jnp.zeros((_SMAX,), jnp.float32).at[slot].set(flat_s)
    starts = jnp.arange(_NB, dtype=jnp.int32) * _BLK
    blk_expert = jnp.clip(
        jnp.searchsorted(pend, starts, side="right"), 0, _E - 1
    ).astype(jnp.int32)
    slot2 = slot.reshape(_T, _K)
    nv = (pend[_E - 1] // _BLK).astype(jnp.int32).reshape(1)
    return slot, slot2[:, 0], slot2[:, 1], ws_flat, blk_expert, nv


def kernel(x, gate_w, Wg, Wu, Wd, sg_w, su_w, sd_w, seg_w):
    inds2, sc2, rank2, cnt = _router(x, gate_w)
    slot, idx0, idx1, ws_flat, blk_expert, nv = _dispatch(inds2, sc2, rank2,
                                                          cnt)
    x_sorted = _scatter_x(x, idx0, idx1)
    y_sorted = _gmm(blk_expert, nv, x_sorted, Wg, Wu, Wd,
                    ws_flat.reshape(_NB, _BLK, 1))
    yg = _gather_y(y_sorted, slot)
    return _combine(x, yg.reshape(_T, _K, _D), sg_w, su_w, sd_w, seg_w)


# speedup vs baseline: 1.3338x; 1.0034x over previous
import functools

import jax
import jax.numpy as jnp
from jax import lax
from jax.experimental import pallas as pl
from jax.experimental.pallas import tpu as pltpu
from jax.experimental.pallas import tpu_sc as plsc

_T, _D, _E, _K, _DFF, _DSH = 2048, 2048, 64, 2, 512, 512
_BLK = 64
_NB = 128
_SMAX = _NB * _BLK
_TB = 512
_NW = 32


def _router_body(x_ref, gw_ref, inds_ref, sc_ref, rank_ref, cnt_ref, run_ref):
    pid = pl.program_id(0)

    @pl.when(pid == 0)
    def _():
        run_ref[...] = jnp.zeros_like(run_ref)

    xb = x_ref[...]
    logits = lax.dot_general(xb, gw_ref[...], (((1,), (1,)), ((), ())),
                             preferred_element_type=jnp.float32)
    m = jnp.max(logits, axis=1, keepdims=True)
    ex = jnp.exp(logits - m)
    p = ex / jnp.sum(ex, axis=1, keepdims=True)
    idx = lax.broadcasted_iota(jnp.int32, p.shape, 1)
    v1 = jnp.max(p, axis=1, keepdims=True)
    i1 = jnp.min(jnp.where(p == v1, idx, _E), axis=1)
    pm = jnp.where(idx == i1[:, None], -jnp.inf, p)
    v2 = jnp.max(pm, axis=1, keepdims=True)
    i2 = jnp.min(jnp.where(pm == v2, idx, _E), axis=1)
    s = v1[:, 0] + v2[:, 0]
    inds_ref[...] = jnp.stack([i1, i2], axis=0)
    sc_ref[...] = jnp.stack([v1[:, 0] / s, v2[:, 0] / s], axis=0)

    eq1 = (idx == i1[:, None]).astype(jnp.float32)
    eq2 = (idx == i2[:, None]).astype(jnp.float32)
    both = eq1 + eq2
    r = lax.broadcasted_iota(jnp.int32, (_TB, _TB), 0)
    c = lax.broadcasted_iota(jnp.int32, (_TB, _TB), 1)
    tri = jnp.where(c < r, 1.0, 0.0).astype(jnp.float32)
    pre = lax.dot_general(tri, both, (((1,), (0,)), ((), ())),
                          preferred_element_type=jnp.float32)
    tot = pre + run_ref[...]
    rank1 = jnp.sum(eq1 * tot, axis=1)
    rank2 = jnp.sum(eq2 * tot, axis=1)
    rank_ref[...] = jnp.stack([rank1, rank2], axis=0).astype(jnp.int32)
    run_new = run_ref[...] + jnp.sum(both, axis=0, keepdims=True)
    run_ref[...] = run_new
    cnt_ref[...] = run_new.astype(jnp.int32)


def _router(x, gate_w):
    return pl.pallas_call(
        _router_body,
        grid=(_T // _TB,),
        in_specs=[
            pl.BlockSpec((_TB, _D), lambda i: (i, 0)),
            pl.BlockSpec((_E, _D), lambda i: (0, 0)),
        ],
        out_specs=[
            pl.BlockSpec((_K, _TB), lambda i: (0, i)),
            pl.BlockSpec((_K, _TB), lambda i: (0, i)),
            pl.BlockSpec((_K, _TB), lambda i: (0, i)),
            pl.BlockSpec((1, _E), lambda i: (0, 0)),
        ],
        out_shape=[
            jax.ShapeDtypeStruct((_K, _T), jnp.int32),
            jax.ShapeDtypeStruct((_K, _T), jnp.float32),
            jax.ShapeDtypeStruct((_K, _T), jnp.int32),
            jax.ShapeDtypeStruct((1, _E), jnp.int32),
        ],
        scratch_shapes=[pltpu.VMEM((1, _E), jnp.float32)],
    )(x, gate_w)


@functools.lru_cache(maxsize=None)
def _make_sc_scatter(chunk):
    per_w = _T // _NW
    n_steps = per_w // chunk
    assert n_steps % 2 == 0
    mesh = plsc.VectorSubcoreMesh(core_axis_name="c", subcore_axis_name="s")

    def body(x_ref, i0_ref, i1_ref, out_ref, idx_v, rows_v,
             sem_l0, sem_l1, sem_s0, sem_s1):
        wid = lax.axis_index("s") * 2 + lax.axis_index("c")
        base = wid * per_w
        sem_l = (sem_l0, sem_l1)
        sem_s = (sem_s0, sem_s1)

        def start_load(i, b):
            off = base + i * chunk
            pltpu.sync_copy(i0_ref.at[pl.ds(off, chunk)], idx_v.at[2 * b])
            pltpu.sync_copy(i1_ref.at[pl.ds(off, chunk)], idx_v.at[2 * b + 1])
            pltpu.async_copy(x_ref.at[pl.ds(off, chunk)], rows_v.at[b],
                             sem_l[b])

        def wait_load(b):
            pltpu.make_async_copy(x_ref.at[pl.ds(base, chunk)], rows_v.at[b],
                                  sem_l[b]).wait()

        def start_scat(b):
            pltpu.async_copy(rows_v.at[b], out_ref.at[idx_v.at[2 * b]],
                             sem_s[b])
            pltpu.async_copy(rows_v.at[b], out_ref.at[idx_v.at[2 * b + 1]],
                             sem_s[b])

        def wait_scat(b):
            pltpu.make_async_copy(rows_v.at[b], out_ref.at[idx_v.at[2 * b]],
                                  sem_s[b]).wait()
            pltpu.make_async_copy(rows_v.at[b], out_ref.at[idx_v.at[2 * b + 1]],
                                  sem_s[b]).wait()

        start_load(0, 0)

        def outer(o, carry):
            for b in (0, 1):
                i = o * 2 + b
                wait_load(b)
                start_scat(b)
                nb = 1 - b

                @pl.when(i + 1 < n_steps)
                def _():
                    @pl.when(i >= 1)
                    def _():
                        wait_scat(nb)
                    start_load(i + 1, nb)
            return carry

        lax.fori_loop(0, n_steps // 2, outer, 0)
        wait_scat(0)
        wait_scat(1)

    return pl.kernel(
        body,
        out_type=jax.ShapeDtypeStruct((_SMAX, _D), jnp.float32),
        mesh=mesh,
        scratch_types=[
            pltpu.VMEM((4, chunk), jnp.int32),
            pltpu.VMEM((2, chunk, _D), jnp.float32),
            pltpu.SemaphoreType.DMA,
            pltpu.SemaphoreType.DMA,
            pltpu.SemaphoreType.DMA,
            pltpu.SemaphoreType.DMA,
        ],
    )


def _scatter_x(x, idx0, idx1):
    return _make_sc_scatter(16)(x, idx0, idx1)


@functools.lru_cache(maxsize=None)
def _make_sc_gather(n_out, width, chunk, dtype):
    per_w = n_out // _NW
    n_steps = per_w // chunk
    assert n_steps % 2 == 0
    mesh = plsc.VectorSubcoreMesh(core_axis_name="c", subcore_axis_name="s")

    def body(table_ref, idx_ref, out_ref, idx_v, rows_v,
             sem_g0, sem_g1, sem_o0, sem_o1):
        wid = lax.axis_index("s") * 2 + lax.axis_index("c")
        base = wid * per_w
        sem_g = (sem_g0, sem_g1)
        sem_o = (sem_o0, sem_o1)

        def start_gather(i, b):
            pltpu.sync_copy(idx_ref.at[pl.ds(base + i * chunk, chunk)],
                            idx_v.at[b])
            pltpu.async_copy(table_ref.at[idx_v.at[b]], rows_v.at[b],
                             sem_g[b])

        def wait_gather(b):
            pltpu.make_async_copy(table_ref.at[idx_v.at[b]], rows_v.at[b],
                                  sem_g[b]).wait()

        def start_out(i, b):
            pltpu.async_copy(rows_v.at[b],
                             out_ref.at[pl.ds(base + i * chunk, chunk)],
                             sem_o[b])

        def wait_out(b):
            pltpu.make_async_copy(rows_v.at[b],
                                  out_ref.at[pl.ds(base, chunk)],
                                  sem_o[b]).wait()

        start_gather(0, 0)

        def outer(o, carry):
            for b in (0, 1):
                i = o * 2 + b
                wait_gather(b)
                start_out(i, b)
                nb = 1 - b

                @pl.when(i + 1 < n_steps)
                def _():
                    @pl.when(i >= 1)
                    def _():
                        wait_out(nb)
                    start_gather(i + 1, nb)
            return carry

        lax.fori_loop(0, n_steps // 2, outer, 0)
        wait_out(0)
        wait_out(1)

    return pl.kernel(
        body,
        out_type=jax.ShapeDtypeStruct((n_out, width), dtype),
        mesh=mesh,
        scratch_types=[
            pltpu.VMEM((2, chunk), jnp.int32),
            pltpu.VMEM((2, chunk, width), dtype),
            pltpu.SemaphoreType.DMA,
            pltpu.SemaphoreType.DMA,
            pltpu.SemaphoreType.DMA,
            pltpu.SemaphoreType.DMA,
        ],
    )


def _gather_y(table, idx):
    return _make_sc_gather(_T * _K, _D, 16, jnp.float32)(table, idx)


def _gmm_body(be_ref, nv_ref, x_ref, wg_ref, wu_ref, wd_ref, ws_ref, out_ref):
    @pl.when(pl.program_id(0) < nv_ref[0])
    def _():
        xb = x_ref[...]
        g = lax.dot_general(xb, wg_ref[0], (((1,), (1,)), ((), ())),
                            preferred_element_type=jnp.float32)
        u = lax.dot_general(xb, wu_ref[0], (((1,), (1,)), ((), ())),
                            preferred_element_type=jnp.float32)
        h = g * jax.nn.sigmoid(g) * u
        o = lax.dot_general(h, wd_ref[0], (((1,), (1,)), ((), ())),
                            preferred_element_type=jnp.float32)
        out_ref[...] = o * ws_ref[0]


def _gmm(blk_expert, nv, x_sorted, Wg, Wu, Wd, ws3):
    def _c(i, nv_ref):
        return jnp.minimum(i, nv_ref[0] - 1)

    grid_spec = pltpu.PrefetchScalarGridSpec(
        num_scalar_prefetch=2,
        grid=(_NB,),
        in_specs=[
            pl.BlockSpec((_BLK, _D), lambda i, be, nv: (_c(i, nv), 0)),
            pl.BlockSpec((1, _DFF, _D),
                         lambda i, be, nv: (be[_c(i, nv)], 0, 0)),
            pl.BlockSpec((1, _DFF, _D),
                         lambda i, be, nv: (be[_c(i, nv)], 0, 0)),
            pl.BlockSpec((1, _D, _DFF),
                         lambda i, be, nv: (be[_c(i, nv)], 0, 0)),
            pl.BlockSpec((1, _BLK, 1), lambda i, be, nv: (_c(i, nv), 0, 0)),
        ],
        out_specs=pl.BlockSpec((_BLK, _D), lambda i, be, nv: (_c(i, nv), 0)),
    )
    return pl.pallas_call(
        _gmm_body,
        grid_spec=grid_spec,
        out_shape=jax.ShapeDtypeStruct((_SMAX, _D), jnp.float32),
    )(blk_expert, nv, x_sorted, Wg, Wu, Wd, ws3)


def _combine_body(x_ref, yg_ref, sg_ref, su_ref, sd_ref, se_ref, out_ref):
    xb = x_ref[...]
    g = lax.dot_general(xb, sg_ref[...], (((1,), (1,)), ((), ())),
                        preferred_element_type=jnp.float32)
    u = lax.dot_general(xb, su_ref[...], (((1,), (1,)), ((), ())),
                        preferred_element_type=jnp.float32)
    sh = g * jax.nn.sigmoid(g) * u
    sy = lax.dot_general(sh, sd_ref[...], (((1,), (1,)), ((), ())),
                         preferred_element_type=jnp.float32)
    gate = jax.nn.sigmoid(
        lax.dot_general(xb, se_ref[...], (((1,), (1,)), ((), ())),
                        preferred_element_type=jnp.float32))
    yg = yg_ref[...]
    out_ref[...] = yg[:, 0, :] + yg[:, 1, :] + gate * sy


def _combine(x, yg3, sg_w, su_w, sd_w, seg_w):
    return pl.pallas_call(
        _combine_body,
        grid=(_T // _TB,),
        in_specs=[
            pl.BlockSpec((_TB, _D), lambda i: (i, 0)),
            pl.BlockSpec((_TB, _K, _D), lambda i: (i, 0, 0)),
            pl.BlockSpec((_DSH, _D), lambda i: (0, 0)),
            pl.BlockSpec((_DSH, _D), lambda i: (0, 0)),
            pl.BlockSpec((_D, _DSH), lambda i: (0, 0)),
            pl.BlockSpec((1, _D), lambda i: (0, 0)),
        ],
        out_specs=pl.BlockSpec((_TB, _D), lambda i: (i, 0)),
        out_shape=jax.ShapeDtypeStruct((_T, _D), jnp.float32),
    )(x, yg3, sg_w, su_w, sd_w, seg_w)


def _dispatch(inds2, sc2, rank2, cnt):
    flat_e = inds2.T.reshape(-1)
    flat_s = sc2.T.reshape(-1)
    rank_flat = rank2.T.reshape(-1)
    counts = cnt[0]
    pcnt = ((counts + _BLK - 1) // _BLK) * _BLK
    pend = jnp.cumsum(pcnt)
    poff = pend - pcnt
    slot = poff[flat_e] + rank_flat
    ws_flat = jnp.zeros((_SMAX,), jnp.float32).at[slot].set(flat_s)
    starts = jnp.arange(_NB, dtype=jnp.int32) * _BLK
    blk_expert = jnp.clip(
        jnp.searchsorted(pend, starts, side="right"), 0, _E - 1
    ).astype(jnp.int32)
    slot2 = slot.reshape(_T, _K)
    nv = (pend[_E - 1] // _BLK).astype(jnp.int32).reshape(1)
    return slot, slot2[:, 0], slot2[:, 1], ws_flat, blk_expert, nv


def kernel(x, gate_w, Wg, Wu, Wd, sg_w, su_w, sd_w, seg_w):
    inds2, sc2, rank2, cnt = _router(x, gate_w)
    slot, idx0, idx1, ws_flat, blk_expert, nv = _dispatch(inds2, sc2, rank2,
                                                          cnt)
    x_sorted = _scatter_x(x, idx0, idx1)
    y_sorted = _gmm(blk_expert, nv, x_sorted, Wg, Wu, Wd,
                    ws_flat.reshape(_NB, _BLK, 1))
    yg = _gather_y(y_sorted, slot)
    return _combine(x, yg.reshape(_T, _K, _D), sg_w, su_w, sd_w, seg_w)


# speedup vs baseline: 1.3341x; 1.0002x over previous
import functools

import jax
import jax.numpy as jnp
from jax import lax
from jax.experimental import pallas as pl
from jax.experimental.pallas import tpu as pltpu
from jax.experimental.pallas import tpu_sc as plsc

_T, _D, _E, _K, _DFF, _DSH = 2048, 2048, 64, 2, 512, 512
_BLK = 64
_NB = 128
_SMAX = _NB * _BLK
_TB = 512
_NW = 32


def _router_body(x_ref, gw_ref, inds_ref, sc_ref, rank_ref, cnt_ref, run_ref):
    pid = pl.program_id(0)

    @pl.when(pid == 0)
    def _():
        run_ref[...] = jnp.zeros_like(run_ref)

    xb = x_ref[...]
    logits = lax.dot_general(xb, gw_ref[...], (((1,), (1,)), ((), ())),
                             preferred_element_type=jnp.float32)
    m = jnp.max(logits, axis=1, keepdims=True)
    ex = jnp.exp(logits - m)
    p = ex / jnp.sum(ex, axis=1, keepdims=True)
    idx = lax.broadcasted_iota(jnp.int32, p.shape, 1)
    v1 = jnp.max(p, axis=1, keepdims=True)
    i1 = jnp.min(jnp.where(p == v1, idx, _E), axis=1)
    pm = jnp.where(idx == i1[:, None], -jnp.inf, p)
    v2 = jnp.max(pm, axis=1, keepdims=True)
    i2 = jnp.min(jnp.where(pm == v2, idx, _E), axis=1)
    s = v1[:, 0] + v2[:, 0]
    inds_ref[...] = jnp.stack([i1, i2], axis=0)
    sc_ref[...] = jnp.stack([v1[:, 0] / s, v2[:, 0] / s], axis=0)

    eq1 = (idx == i1[:, None]).astype(jnp.float32)
    eq2 = (idx == i2[:, None]).astype(jnp.float32)
    both = eq1 + eq2
    r = lax.broadcasted_iota(jnp.int32, (_TB, _TB), 0)
    c = lax.broadcasted_iota(jnp.int32, (_TB, _TB), 1)
    tri = jnp.where(c < r, 1.0, 0.0).astype(jnp.float32)
    pre = lax.dot_general(tri, both, (((1,), (0,)), ((), ())),
                          preferred_element_type=jnp.float32)
    tot = pre + run_ref[...]
    rank1 = jnp.sum(eq1 * tot, axis=1)
    rank2 = jnp.sum(eq2 * tot, axis=1)
    rank_ref[...] = (jnp.stack([rank1, rank2], axis=0) + 0.5).astype(jnp.int32)
    run_new = run_ref[...] + jnp.sum(both, axis=0, keepdims=True)
    run_ref[...] = run_new
    cnt_ref[...] = run_new.astype(jnp.int32)


def _router(x, gate_w):
    return pl.pallas_call(
        _router_body,
        grid=(_T // _TB,),
        in_specs=[
            pl.BlockSpec((_TB, _D), lambda i: (i, 0)),
            pl.BlockSpec((_E, _D), lambda i: (0, 0)),
        ],
        out_specs=[
            pl.BlockSpec((_K, _TB), lambda i: (0, i)),
            pl.BlockSpec((_K, _TB), lambda i: (0, i)),
            pl.BlockSpec((_K, _TB), lambda i: (0, i)),
            pl.BlockSpec((1, _E), lambda i: (0, 0)),
        ],
        out_shape=[
            jax.ShapeDtypeStruct((_K, _T), jnp.int32),
            jax.ShapeDtypeStruct((_K, _T), jnp.float32),
            jax.ShapeDtypeStruct((_K, _T), jnp.int32),
            jax.ShapeDtypeStruct((1, _E), jnp.int32),
        ],
        scratch_shapes=[pltpu.VMEM((1, _E), jnp.float32)],
    )(x, gate_w)


@functools.lru_cache(maxsize=None)
def _make_sc_scatter(chunk):
    per_w = _T // _NW
    n_steps = per_w // chunk
    assert n_steps % 2 == 0
    mesh = plsc.VectorSubcoreMesh(core_axis_name="c", subcore_axis_name="s")

    def body(x_ref, i0_ref, i1_ref, out_ref, idx_v, rows_v,
             sem_l0, sem_l1, sem_s0, sem_s1):
        wid = lax.axis_index("s") * 2 + lax.axis_index("c")
        base = wid * per_w
        sem_l = (sem_l0, sem_l1)
        sem_s = (sem_s0, sem_s1)

        def start_load(i, b):
            off = base + i * chunk
            pltpu.sync_copy(i0_ref.at[pl.ds(off, chunk)], idx_v.at[2 * b])
            pltpu.sync_copy(i1_ref.at[pl.ds(off, chunk)], idx_v.at[2 * b + 1])
            pltpu.async_copy(x_ref.at[pl.ds(off, chunk)], rows_v.at[b],
                             sem_l[b])

        def wait_load(b):
            pltpu.make_async_copy(x_ref.at[pl.ds(base, chunk)], rows_v.at[b],
                                  sem_l[b]).wait()

        def start_scat(b):
            pltpu.async_copy(rows_v.at[b], out_ref.at[idx_v.at[2 * b]],
                             sem_s[b])
            pltpu.async_copy(rows_v.at[b], out_ref.at[idx_v.at[2 * b + 1]],
                             sem_s[b])

        def wait_scat(b):
            pltpu.make_async_copy(rows_v.at[b], out_ref.at[idx_v.at[2 * b]],
                                  sem_s[b]).wait()
            pltpu.make_async_copy(rows_v.at[b], out_ref.at[idx_v.at[2 * b + 1]],
                                  sem_s[b]).wait()

        start_load(0, 0)

        def outer(o, carry):
            for b in (0, 1):
                i = o * 2 + b
                wait_load(b)
                start_scat(b)
                nb = 1 - b

                @pl.when(i + 1 < n_steps)
                def _():
                    @pl.when(i >= 1)
                    def _():
                        wait_scat(nb)
                    start_load(i + 1, nb)
            return carry

        lax.fori_loop(0, n_steps // 2, outer, 0)
        wait_scat(0)
        wait_scat(1)

    return pl.kernel(
        body,
        out_type=jax.ShapeDtypeStruct((_SMAX, _D), jnp.float32),
        mesh=mesh,
        scratch_types=[
            pltpu.VMEM((4, chunk), jnp.int32),
            pltpu.VMEM((2, chunk, _D), jnp.float32),
            pltpu.SemaphoreType.DMA,
            pltpu.SemaphoreType.DMA,
            pltpu.SemaphoreType.DMA,
            pltpu.SemaphoreType.DMA,
        ],
    )


def _scatter_x(x, idx0, idx1):
    return _make_sc_scatter(16)(x, idx0, idx1)


@functools.lru_cache(maxsize=None)
def _make_sc_gather(n_out, width, chunk, dtype):
    per_w = n_out // _NW
    n_steps = per_w // chunk
    assert n_steps % 2 == 0
    mesh = plsc.VectorSubcoreMesh(core_axis_name="c", subcore_axis_name="s")

    def body(table_ref, idx_ref, out_ref, idx_v, rows_v,
             sem_g0, sem_g1, sem_o0, sem_o1):
        wid = lax.axis_index("s") * 2 + lax.axis_index("c")
        base = wid * per_w
        sem_g = (sem_g0, sem_g1)
        sem_o = (sem_o0, sem_o1)

        def start_gather(i, b):
            pltpu.sync_copy(idx_ref.at[pl.ds(base + i * chunk, chunk)],
                            idx_v.at[b])
            pltpu.async_copy(table_ref.at[idx_v.at[b]], rows_v.at[b],
                             sem_g[b])

        def wait_gather(b):
            pltpu.make_async_copy(table_ref.at[idx_v.at[b]], rows_v.at[b],
                                  sem_g[b]).wait()

        def start_out(i, b):
            pltpu.async_copy(rows_v.at[b],
                             out_ref.at[pl.ds(base + i * chunk, chunk)],
                             sem_o[b])

        def wait_out(b):
            pltpu.make_async_copy(rows_v.at[b],
                                  out_ref.at[pl.ds(base, chunk)],
                                  sem_o[b]).wait()

        start_gather(0, 0)

        def outer(o, carry):
            for b in (0, 1):
                i = o * 2 + b
                wait_gather(b)
                start_out(i, b)
                nb = 1 - b

                @pl.when(i + 1 < n_steps)
                def _():
                    @pl.when(i >= 1)
                    def _():
                        wait_out(nb)
                    start_gather(i + 1, nb)
            return carry

        lax.fori_loop(0, n_steps // 2, outer, 0)
        wait_out(0)
        wait_out(1)

    return pl.kernel(
        body,
        out_type=jax.ShapeDtypeStruct((n_out, width), dtype),
        mesh=mesh,
        scratch_types=[
            pltpu.VMEM((2, chunk), jnp.int32),
            pltpu.VMEM((2, chunk, width), dtype),
            pltpu.SemaphoreType.DMA,
            pltpu.SemaphoreType.DMA,
            pltpu.SemaphoreType.DMA,
            pltpu.SemaphoreType.DMA,
        ],
    )


def _gather_y(table, idx):
    return _make_sc_gather(_T * _K, _D, 16, jnp.float32)(table, idx)


def _gmm_body(be_ref, nv_ref, x_ref, wg_ref, wu_ref, wd_ref, ws_ref, out_ref):
    @pl.when(pl.program_id(0) < nv_ref[0])
    def _():
        xb = x_ref[...]
        g = lax.dot_general(xb, wg_ref[0], (((1,), (1,)), ((), ())),
                            preferred_element_type=jnp.float32)
        u = lax.dot_general(xb, wu_ref[0], (((1,), (1,)), ((), ())),
                            preferred_element_type=jnp.float32)
        h = g * jax.nn.sigmoid(g) * u
        o = lax.dot_general(h, wd_ref[0], (((1,), (1,)), ((), ())),
                            preferred_element_type=jnp.float32)
        out_ref[...] = o * ws_ref[0]


def _gmm(blk_expert, nv, x_sorted, Wg, Wu, Wd, ws3):
    def _c(i, nv_ref):
        return jnp.minimum(i, nv_ref[0] - 1)

    grid_spec = pltpu.PrefetchScalarGridSpec(
        num_scalar_prefetch=2,
        grid=(_NB,),
        in_specs=[
            pl.BlockSpec((_BLK, _D), lambda i, be, nv: (_c(i, nv), 0)),
            pl.BlockSpec((1, _DFF, _D),
                         lambda i, be, nv: (be[_c(i, nv)], 0, 0)),
            pl.BlockSpec((1, _DFF, _D),
                         lambda i, be, nv: (be[_c(i, nv)], 0, 0)),
            pl.BlockSpec((1, _D, _DFF),
                         lambda i, be, nv: (be[_c(i, nv)], 0, 0)),
            pl.BlockSpec((1, _BLK, 1), lambda i, be, nv: (_c(i, nv), 0, 0)),
        ],
        out_specs=pl.BlockSpec((_BLK, _D), lambda i, be, nv: (_c(i, nv), 0)),
    )
    return pl.pallas_call(
        _gmm_body,
        grid_spec=grid_spec,
        out_shape=jax.ShapeDtypeStruct((_SMAX, _D), jnp.float32),
    )(blk_expert, nv, x_sorted, Wg, Wu, Wd, ws3)


def _combine_body(x_ref, yg_ref, sg_ref, su_ref, sd_ref, se_ref, out_ref):
    xb = x_ref[...]
    g = lax.dot_general(xb, sg_ref[...], (((1,), (1,)), ((), ())),
                        preferred_element_type=jnp.float32)
    u = lax.dot_general(xb, su_ref[...], (((1,), (1,)), ((), ())),
                        preferred_element_type=jnp.float32)
    sh = g * jax.nn.sigmoid(g) * u
    sy = lax.dot_general(sh, sd_ref[...], (((1,), (1,)), ((), ())),
                         preferred_element_type=jnp.float32)
    gate = jax.nn.sigmoid(
        lax.dot_general(xb, se_ref[...], (((1,), (1,)), ((), ())),
                        preferred_element_type=jnp.float32))
    yg = yg_ref[...]
    out_ref[...] = yg[:, 0, :] + yg[:, 1, :] + gate * sy


def _combine(x, yg3, sg_w, su_w, sd_w, seg_w):
    return pl.pallas_call(
        _combine_body,
        grid=(_T // _TB,),
        in_specs=[
            pl.BlockSpec((_TB, _D), lambda i: (i, 0)),
            pl.BlockSpec((_TB, _K, _D), lambda i: (i, 0, 0)),
            pl.BlockSpec((_DSH, _D), lambda i: (0, 0)),
            pl.BlockSpec((_DSH, _D), lambda i: (0, 0)),
            pl.BlockSpec((_D, _DSH), lambda i: (0, 0)),
            pl.BlockSpec((1, _D), lambda i: (0, 0)),
        ],
        out_specs=pl.BlockSpec((_TB, _D), lambda i: (i, 0)),
        out_shape=jax.ShapeDtypeStruct((_T, _D), jnp.float32),
    )(x, yg3, sg_w, su_w, sd_w, seg_w)


def _dispatch(inds2, sc2, rank2, cnt):
    flat_e = inds2.T.reshape(-1)
    flat_s = sc2.T.reshape(-1)
    rank_flat = rank2.T.reshape(-1)
    counts = cnt[0]
    pcnt = ((counts + _BLK - 1) // _BLK) * _BLK
    pend = jnp.cumsum(pcnt)
    poff = pend - pcnt
    slot = poff[flat_e] + rank_flat
    ws_flat = jnp.zeros((_SMAX,), jnp.float32).at[slot].set(flat_s)
    starts = jnp.arange(_NB, dtype=jnp.int32) * _BLK
    blk_expert = jnp.clip(
        jnp.searchsorted(pend, starts, side="right"), 0, _E - 1
    ).astype(jnp.int32)
    slot2 = slot.reshape(_T, _K)
    nv = (pend[_E - 1] // _BLK).astype(jnp.int32).reshape(1)
    return slot, slot2[:, 0], slot2[:, 1], ws_flat, blk_expert, nv


def kernel(x, gate_w, Wg, Wu, Wd, sg_w, su_w, sd_w, seg_w):
    inds2, sc2, rank2, cnt = _router(x, gate_w)
    slot, idx0, idx1, ws_flat, blk_expert, nv = _dispatch(inds2, sc2, rank2,
                                                          cnt)
    x_sorted = _scatter_x(x, idx0, idx1)
    y_sorted = _gmm(blk_expert, nv, x_sorted, Wg, Wu, Wd,
                    ws_flat.reshape(_NB, _BLK, 1))
    yg = _gather_y(y_sorted, slot)
    return _combine(x, yg.reshape(_T, _K, _D), sg_w, su_w, sd_w, seg_w)
